# Initial kernel scaffold; baseline (speedup 1.0000x reference)
import jax
import jax.numpy as jnp
from jax.experimental import pallas as pl


def kernel(x_user, x_item, edge_index_buys, edge_embed_buys, edge_index_rev, edge_embed_rev, Wk_user, bk_user, Wv_user, bv_user, Wq_user, bq_user, Wk_item, bk_item, Wv_item, bv_item, Wq_item, bq_item, We_buys, be_buys, We_rev, be_rev, Wout, bout, g_user, b_ln_user, g_item, b_ln_item):
    raise NotImplementedError("write your pallas kernel here")


# speedup vs baseline: 6.7955x; 6.7955x over previous
import functools

import jax
import jax.numpy as jnp
from jax import lax
from jax.experimental import pallas as pl
from jax.experimental.pallas import tpu as pltpu

N = 50000
D = 128
H = 4
HD = 32
E = 200000
DE = 16

E_PAD = 200704
BM = 5000
BE = 512


def _proj_body(x_ref, w_ref, b_ref, o_ref):
    o_ref[0] = (
        jnp.dot(x_ref[0], w_ref[0], preferred_element_type=jnp.float32)
        + b_ref[0]
    )


def _proj(xs, ws, bs):
    return pl.pallas_call(
        _proj_body,
        grid=(2, N // BM),
        in_specs=[
            pl.BlockSpec((1, BM, D), lambda t, i: (t, i, 0)),
            pl.BlockSpec((1, D, 3 * D), lambda t, i: (t, 0, 0)),
            pl.BlockSpec((1, 1, 3 * D), lambda t, i: (t, 0, 0)),
        ],
        out_specs=pl.BlockSpec((1, BM, 3 * D), lambda t, i: (t, i, 0)),
        out_shape=jax.ShapeDtypeStruct((2, N, 3 * D), jnp.float32),
    )(xs, ws, bs)


def _edge_body(kg_ref, qg_ref, vg_ref, ee_ref, we_ref, be_ref, m16_ref,
               wv_ref, ex_ref):
    ke = jnp.dot(ee_ref[...], we_ref[...],
                 preferred_element_type=jnp.float32) + be_ref[...]
    t = (kg_ref[...] + ke) * qg_ref[...]
    s16 = jnp.dot(t, m16_ref[...], preferred_element_type=jnp.float32)
    ex16 = jnp.exp(s16 * (1.0 / (HD ** 0.5)))
    ex_ref[...] = ex16
    exb = lax.dot_general(ex16, m16_ref[...], (((1,), (1,)), ((), ())),
                          preferred_element_type=jnp.float32)
    wv_ref[...] = vg_ref[...] * exb


def _edge_stage(kg, qg, vg, eemb, we, be, m16):
    return pl.pallas_call(
        _edge_body,
        grid=(E_PAD // BE,),
        in_specs=[
            pl.BlockSpec((BE, D), lambda i: (i, 0)),
            pl.BlockSpec((BE, D), lambda i: (i, 0)),
            pl.BlockSpec((BE, D), lambda i: (i, 0)),
            pl.BlockSpec((BE, DE), lambda i: (i, 0)),
            pl.BlockSpec((DE, D), lambda i: (0, 0)),
            pl.BlockSpec((1, D), lambda i: (0, 0)),
            pl.BlockSpec((D, 16), lambda i: (0, 0)),
        ],
        out_specs=[
            pl.BlockSpec((BE, D), lambda i: (i, 0)),
            pl.BlockSpec((BE, 16), lambda i: (i, 0)),
        ],
        out_shape=[
            jax.ShapeDtypeStruct((E_PAD, D), jnp.float32),
            jax.ShapeDtypeStruct((E_PAD, 16), jnp.float32),
        ],
    )(kg, qg, vg, eemb, we, be, m16)


def _final_body(x_ref, mu_ref, ss_ref, m16_ref, wo_ref, bo_ref, g_ref,
                b_ref, o_ref):
    ssb = lax.dot_general(ss_ref[...], m16_ref[...], (((1,), (1,)), ((), ())),
                          preferred_element_type=jnp.float32)
    m = mu_ref[...] / (ssb + 1e-16)
    z = x_ref[...] + jnp.dot(m, wo_ref[...],
                             preferred_element_type=jnp.float32) + bo_ref[...]
    mu = jnp.mean(z, axis=-1, keepdims=True)
    zc = z - mu
    var = jnp.mean(zc * zc, axis=-1, keepdims=True)
    y = zc * lax.rsqrt(var + 1e-5) * g_ref[...] + b_ref[...]
    o_ref[...] = jnp.where(y >= 0, y, 0.2 * y)


def _final_stage(x, msgU, ssum, m16, wout, bout, g, b):
    return pl.pallas_call(
        _final_body,
        grid=(N // BM,),
        in_specs=[
            pl.BlockSpec((BM, D), lambda i: (i, 0)),
            pl.BlockSpec((BM, D), lambda i: (i, 0)),
            pl.BlockSpec((BM, 16), lambda i: (i, 0)),
            pl.BlockSpec((D, 16), lambda i: (0, 0)),
            pl.BlockSpec((D, D), lambda i: (0, 0)),
            pl.BlockSpec((1, D), lambda i: (0, 0)),
            pl.BlockSpec((1, D), lambda i: (0, 0)),
            pl.BlockSpec((1, D), lambda i: (0, 0)),
        ],
        out_specs=pl.BlockSpec((BM, D), lambda i: (i, 0)),
        out_shape=jax.ShapeDtypeStruct((N, D), jnp.float32),
    )(x, msgU, ssum, m16, wout, bout, g, b)


def _head_mask():
    d = jnp.arange(D)[:, None]
    h = jnp.arange(16)[None, :]
    return (d // HD == h).astype(jnp.float32)


def kernel(x_user, x_item, edge_index_buys, edge_embed_buys, edge_index_rev,
           edge_embed_rev, Wk_user, bk_user, Wv_user, bv_user, Wq_user,
           bq_user, Wk_item, bk_item, Wv_item, bv_item, Wq_item, bq_item,
           We_buys, be_buys, We_rev, be_rev, Wout, bout, g_user, b_ln_user,
           g_item, b_ln_item):
    xs = jnp.stack([x_user, x_item])
    ws = jnp.stack([
        jnp.concatenate([Wk_user, Wv_user, Wq_user], axis=1),
        jnp.concatenate([Wk_item, Wv_item, Wq_item], axis=1),
    ])
    bs = jnp.stack([
        jnp.concatenate([bk_user, bv_user, bq_user])[None, :],
        jnp.concatenate([bk_item, bv_item, bq_item])[None, :],
    ])
    kvq = _proj(xs, ws, bs)
    K_u, V_u, Q_u = kvq[0, :, :D], kvq[0, :, D:2 * D], kvq[0, :, 2 * D:]
    K_i, V_i, Q_i = kvq[1, :, :D], kvq[1, :, D:2 * D], kvq[1, :, 2 * D:]

    m16 = _head_mask()

    def rel(K_src, V_src, Q_dst, eidx, eemb, We, be):
        src = jnp.concatenate(
            [eidx[0], jnp.zeros((E_PAD - E,), eidx.dtype)])
        dst = jnp.concatenate(
            [eidx[1], jnp.full((E_PAD - E,), N, eidx.dtype)])
        kg = K_src[src]
        vg = V_src[src]
        qg = Q_dst[dst.clip(0, N - 1)]
        eembp = jnp.concatenate(
            [eemb, jnp.zeros((E_PAD - E, DE), jnp.float32)], axis=0)
        wv, ex16 = _edge_stage(kg, qg, vg, eembp, We, be[None, :], m16)
        ssum = jax.ops.segment_sum(ex16, dst, num_segments=N + 1)[:N]
        msgU = jax.ops.segment_sum(wv, dst, num_segments=N + 1)[:N]
        return msgU, ssum

    msgU_i, ssum_i = rel(K_u, V_u, Q_i, edge_index_buys, edge_embed_buys,
                         We_buys, be_buys)
    msgU_u, ssum_u = rel(K_i, V_i, Q_u, edge_index_rev, edge_embed_rev,
                         We_rev, be_rev)

    out_u = _final_stage(x_user, msgU_u, ssum_u, m16, Wout, bout[None, :],
                         g_user[None, :], b_ln_user[None, :])
    out_i = _final_stage(x_item, msgU_i, ssum_i, m16, Wout, bout[None, :],
                         g_item[None, :], b_ln_item[None, :])
    return jnp.stack([out_u, out_i], axis=0)


# speedup vs baseline: 11.2553x; 1.6563x over previous
import functools

import jax
import jax.numpy as jnp
from jax import lax
from jax.experimental import pallas as pl
from jax.experimental.pallas import tpu as pltpu
from jax.experimental.pallas import tpu_sc as plsc

N = 50000
D = 128
H = 4
HD = 32
E = 200000
DE = 16

N_PAD = 50176
E_PAD = 200704
BM = 5000
BMP = 6272
BE = 512

CH = 128
NSC = 2
NTEC = 16
NW = NSC * NTEC
EPW = E_PAD // NW
EPT = E_PAD // NTEC
RNG = N_PAD // 8
RROWS = 6400


def _sc_mesh():
    return plsc.VectorSubcoreMesh(core_axis_name="c", subcore_axis_name="s")


def _proj_body(x_ref, w_ref, b_ref, k_ref, v_ref, q_ref):
    r = (jnp.dot(x_ref[0], w_ref[0], preferred_element_type=jnp.float32)
         + b_ref[0])
    k_ref[0] = r[:, :D]
    v_ref[0] = r[:, D:2 * D]
    q_ref[0] = r[:, 2 * D:]


def _proj(xs, ws, bs):
    return pl.pallas_call(
        _proj_body,
        grid=(2, N_PAD // BMP),
        in_specs=[
            pl.BlockSpec((1, BMP, D), lambda t, i: (t, i, 0)),
            pl.BlockSpec((1, D, 3 * D), lambda t, i: (t, 0, 0)),
            pl.BlockSpec((1, 1, 3 * D), lambda t, i: (t, 0, 0)),
        ],
        out_specs=[
            pl.BlockSpec((1, BMP, D), lambda t, i: (t, i, 0)),
            pl.BlockSpec((1, BMP, D), lambda t, i: (t, i, 0)),
            pl.BlockSpec((1, BMP, D), lambda t, i: (t, i, 0)),
        ],
        out_shape=[
            jax.ShapeDtypeStruct((2, N_PAD, D), jnp.float32),
            jax.ShapeDtypeStruct((2, N_PAD, D), jnp.float32),
            jax.ShapeDtypeStruct((2, N_PAD, D), jnp.float32),
        ],
    )(xs, ws, bs)


def _gather_body(ktab, vtab, qtab, srck, dstq, kg, vg, qg,
                 sidx, didx, kbuf, vbuf, qbuf, sk, sv, sq):
    cid = lax.axis_index("c")
    sid = lax.axis_index("s")
    base = (sid * NSC + cid) * EPW

    def chunk(i, _):
        off = base + i * CH
        pltpu.sync_copy(srck.at[pl.ds(off, CH)], sidx)
        pltpu.sync_copy(dstq.at[pl.ds(off, CH)], didx)
        ck = pltpu.async_copy(ktab.at[sidx], kbuf, sk)
        cv = pltpu.async_copy(vtab.at[sidx], vbuf, sv)
        cq = pltpu.async_copy(qtab.at[didx], qbuf, sq)
        ck.wait()
        cv.wait()
        cq.wait()
        pltpu.sync_copy(kbuf, kg.at[pl.ds(off, CH)])
        pltpu.sync_copy(vbuf, vg.at[pl.ds(off, CH)])
        pltpu.sync_copy(qbuf, qg.at[pl.ds(off, CH)])
        return 0

    lax.fori_loop(0, EPW // CH, chunk, 0)


def _gather(ktab, vtab, qtab, srck, dstq):
    f = pl.kernel(
        _gather_body,
        out_type=[
            jax.ShapeDtypeStruct((E_PAD, D), jnp.float32),
            jax.ShapeDtypeStruct((E_PAD, D), jnp.float32),
            jax.ShapeDtypeStruct((E_PAD, D), jnp.float32),
        ],
        mesh=_sc_mesh(),
        scratch_types=[
            pltpu.VMEM((CH,), jnp.int32),
            pltpu.VMEM((CH,), jnp.int32),
            pltpu.VMEM((CH, D), jnp.float32),
            pltpu.VMEM((CH, D), jnp.float32),
            pltpu.VMEM((CH, D), jnp.float32),
            pltpu.SemaphoreType.DMA,
            pltpu.SemaphoreType.DMA,
            pltpu.SemaphoreType.DMA,
        ],
    )
    return f(ktab, vtab, qtab, srck, dstq)


def _scatter_body(wv0, dst0, wv1, dst1, zwv,
                  msgU_hbm,
                  didx, lidx, aidx, wvbuf, sw, accw):
    cid = lax.axis_index("c")
    sid = lax.axis_index("s")
    tbase = sid * EPT
    lane = jnp.arange(16, dtype=jnp.int32)

    for r in range(2):
        wv_hbm = (wv0, wv1)[r]
        dst_hbm = (dst0, dst1)[r]
        for p in range(4):
            rbase = (cid * 4 + p) * RNG

            @pl.when(sid == 0)
            def _zero():
                pltpu.sync_copy(zwv, accw)

            plsc.subcore_barrier()

            def chunk(i, _):
                off = tbase + i * CH
                pltpu.sync_copy(dst_hbm.at[pl.ds(off, CH)], didx)
                for j in range(CH // 16):
                    dv = didx[pl.ds(j * 16, 16)]
                    offv = dv - rbase
                    ok = (offv >= 0) & (offv < RNG)
                    aidx[0, pl.ds(j * 16, 16)] = jnp.where(ok, offv, RNG)
                    lidx[pl.ds(j * 16, 16)] = off + j * 16 + lane
                pltpu.async_copy(wv_hbm.at[lidx], wvbuf, sw).wait()
                pltpu.sync_copy(wvbuf, accw.at[aidx.at[0]], add=True)
                return 0

            lax.fori_loop(0, EPT // CH, chunk, 0)
            plsc.subcore_barrier()

            @pl.when(sid == 0)
            def _wout():
                pltpu.sync_copy(accw.at[pl.ds(0, RNG)],
                                msgU_hbm.at[r, pl.ds(rbase, RNG)])

            plsc.subcore_barrier()


def _scatter_all(wv0, dst0, wv1, dst1, zwv):
    f = pl.kernel(
        _scatter_body,
        out_type=jax.ShapeDtypeStruct((2, N_PAD, D), jnp.float32),
        mesh=_sc_mesh(),
        scratch_types=[
            pltpu.VMEM((CH,), jnp.int32),
            pltpu.VMEM((CH,), jnp.int32),
            pltpu.VMEM((1, CH), jnp.int32),
            pltpu.VMEM((CH, D), jnp.float32),
            pltpu.SemaphoreType.DMA,
            pltpu.VMEM_SHARED((RROWS, D), jnp.float32),
        ],
    )
    return f(wv0, dst0, wv1, dst1, zwv)


def _edge_body(kg_ref, qg_ref, vg_ref, ee_ref, we_ref, be_ref, m16_ref,
               wv_ref, ex_ref):
    ke = jnp.dot(ee_ref[...], we_ref[...],
                 preferred_element_type=jnp.float32) + be_ref[...]
    t = (kg_ref[...] + ke) * qg_ref[...]
    s16 = jnp.dot(t, m16_ref[...], preferred_element_type=jnp.float32)
    ex16 = jnp.exp(s16 * (1.0 / (HD ** 0.5)))
    ex_ref[...] = ex16
    exb = lax.dot_general(ex16, m16_ref[...], (((1,), (1,)), ((), ())),
                          preferred_element_type=jnp.float32)
    wv_ref[...] = vg_ref[...] * exb


def _edge_stage(kg, qg, vg, eemb, we, be, m16):
    return pl.pallas_call(
        _edge_body,
        grid=(E_PAD // BE,),
        in_specs=[
            pl.BlockSpec((BE, D), lambda i: (i, 0)),
            pl.BlockSpec((BE, D), lambda i: (i, 0)),
            pl.BlockSpec((BE, D), lambda i: (i, 0)),
            pl.BlockSpec((BE, DE), lambda i: (i, 0)),
            pl.BlockSpec((DE, D), lambda i: (0, 0)),
            pl.BlockSpec((1, D), lambda i: (0, 0)),
            pl.BlockSpec((D, 16), lambda i: (0, 0)),
        ],
        out_specs=[
            pl.BlockSpec((BE, D), lambda i: (i, 0)),
            pl.BlockSpec((BE, 16), lambda i: (i, 0)),
        ],
        out_shape=[
            jax.ShapeDtypeStruct((E_PAD, D), jnp.float32),
            jax.ShapeDtypeStruct((E_PAD, 16), jnp.float32),
        ],
    )(kg, qg, vg, eemb, we, be, m16)


def _final_body(x_ref, mu_ref, ss_ref, m16_ref, wo_ref, bo_ref, g_ref,
                b_ref, o_ref):
    ssb = lax.dot_general(ss_ref[0], m16_ref[...], (((1,), (1,)), ((), ())),
                          preferred_element_type=jnp.float32)
    m = mu_ref[0] / (ssb + 1e-16)
    z = x_ref[...] + jnp.dot(m, wo_ref[...],
                             preferred_element_type=jnp.float32) + bo_ref[...]
    mu = jnp.mean(z, axis=-1, keepdims=True)
    zc = z - mu
    var = jnp.mean(zc * zc, axis=-1, keepdims=True)
    y = zc * lax.rsqrt(var + 1e-5) * g_ref[...] + b_ref[...]
    o_ref[...] = jnp.where(y >= 0, y, 0.2 * y)


def _final_stage(x, msgU, ssum, r, m16, wout, bout, g, b):
    return pl.pallas_call(
        _final_body,
        grid=(N // BM,),
        in_specs=[
            pl.BlockSpec((BM, D), lambda i: (i, 0)),
            pl.BlockSpec((1, BM, D), lambda i, r=r: (r, i, 0)),
            pl.BlockSpec((1, BM, 16), lambda i, r=r: (r, i, 0)),
            pl.BlockSpec((D, 16), lambda i: (0, 0)),
            pl.BlockSpec((D, D), lambda i: (0, 0)),
            pl.BlockSpec((1, D), lambda i: (0, 0)),
            pl.BlockSpec((1, D), lambda i: (0, 0)),
            pl.BlockSpec((1, D), lambda i: (0, 0)),
        ],
        out_specs=pl.BlockSpec((BM, D), lambda i: (i, 0)),
        out_shape=jax.ShapeDtypeStruct((N, D), jnp.float32),
    )(x, msgU, ssum, m16, wout, bout, g, b)


def _head_mask():
    d = jnp.arange(D)[:, None]
    h = jnp.arange(16)[None, :]
    return (d // HD == h).astype(jnp.float32)


def kernel(x_user, x_item, edge_index_buys, edge_embed_buys, edge_index_rev,
           edge_embed_rev, Wk_user, bk_user, Wv_user, bv_user, Wq_user,
           bq_user, Wk_item, bk_item, Wv_item, bv_item, Wq_item, bq_item,
           We_buys, be_buys, We_rev, be_rev, Wout, bout, g_user, b_ln_user,
           g_item, b_ln_item):
    xs = jnp.zeros((2, N_PAD, D), jnp.float32)
    xs = xs.at[0, :N].set(x_user).at[1, :N].set(x_item)
    ws = jnp.stack([
        jnp.concatenate([Wk_user, Wv_user, Wq_user], axis=1),
        jnp.concatenate([Wk_item, Wv_item, Wq_item], axis=1),
    ])
    bs = jnp.stack([
        jnp.concatenate([bk_user, bv_user, bq_user])[None, :],
        jnp.concatenate([bk_item, bv_item, bq_item])[None, :],
    ])
    K, V, Q = _proj(xs, ws, bs)
    ktab = K.reshape(2 * N_PAD, D)
    vtab = V.reshape(2 * N_PAD, D)
    qtab = Q.reshape(2 * N_PAD, D)

    m16 = _head_mask()
    zwv = jnp.zeros((RROWS, D), jnp.float32)
    zex = jnp.zeros((RROWS, 16), jnp.float32)

    def rel_edges(src_t, dst_t, eidx, eemb, We, be):
        src = jnp.concatenate([eidx[0], jnp.zeros((E_PAD - E,), eidx.dtype)])
        dst = jnp.concatenate([eidx[1], jnp.full((E_PAD - E,), N, eidx.dtype)])
        srck = src + src_t * N_PAD
        dstq = dst + dst_t * N_PAD
        kg, vg, qg = _gather(ktab, vtab, qtab, srck, dstq)
        eembp = jnp.concatenate(
            [eemb, jnp.zeros((E_PAD - E, DE), jnp.float32)], axis=0)
        wv, ex16 = _edge_stage(kg, qg, vg, eembp, We, be[None, :], m16)
        return wv, ex16, dst

    wv_b, ex_b, dst_b = rel_edges(0, 1, edge_index_buys, edge_embed_buys,
                                  We_buys, be_buys)
    wv_r, ex_r, dst_r = rel_edges(1, 0, edge_index_rev, edge_embed_rev,
                                  We_rev, be_rev)
    msgU = _scatter_all(wv_b, dst_b, wv_r, dst_r, zwv)
    ssum = jnp.stack([
        jax.ops.segment_sum(ex_b, dst_b, num_segments=N_PAD),
        jax.ops.segment_sum(ex_r, dst_r, num_segments=N_PAD)])

    out_u = _final_stage(x_user, msgU, ssum, 1, m16, Wout, bout[None, :],
                         g_user[None, :], b_ln_user[None, :])
    out_i = _final_stage(x_item, msgU, ssum, 0, m16, Wout, bout[None, :],
                         g_item[None, :], b_ln_item[None, :])
    return jnp.stack([out_u, out_i], axis=0)
